# Initial kernel scaffold; baseline (speedup 1.0000x reference)
import functools

import jax
import jax.numpy as jnp
from jax import lax
from jax.experimental import pallas as pl
from jax.experimental.pallas import tpu as pltpu
from jax.experimental.pallas import tpu_sc as plsc

N_NODES = 100000
N_EDGES = 6400000
NBATCH = 16

NC = 2
NS = 16
L = 16
NW = NC * NS
EW = N_EDGES // NW
CHUNK = 80
NCHUNK = EW // CHUNK

NPAD = 100096
RPT = NPAD // NS


@functools.partial(
    pl.kernel,
    out_type=[
        jax.ShapeDtypeStruct((NC * NPAD, 3), jnp.float32),
        jax.ShapeDtypeStruct((NC * NPAD, 3), jnp.float32),
        jax.ShapeDtypeStruct((NC * NPAD, 6), jnp.float32),
    ],
    mesh=plsc.VectorSubcoreMesh(core_axis_name="c", subcore_axis_name="s"),
    scratch_types=[
        pltpu.VMEM((CHUNK, 3), jnp.float32),
        pltpu.VMEM((CHUNK,), jnp.int32),
        pltpu.VMEM((CHUNK,), jnp.int32),
        pltpu.VMEM((CHUNK, 6), jnp.float32),
        pltpu.VMEM_SHARED((NPAD, 3), jnp.float32),
        pltpu.VMEM_SHARED((NPAD, 3), jnp.float32),
        pltpu.VMEM_SHARED((NPAD, 6), jnp.float32),
    ],
)
def _sc_scatter(rij_hbm, src_hbm, dst_hbm, z3_hbm, z6_hbm,
                outA, outB, outC, rij_v, src_v, dst_v, vir_v, tA, tB, tC):
    cid = lax.axis_index("c")
    sid = lax.axis_index("s")
    wid = sid * NC + cid

    r0 = sid * RPT
    pltpu.sync_copy(z3_hbm.at[pl.ds(r0, RPT)], tA.at[pl.ds(r0, RPT)])
    pltpu.sync_copy(z3_hbm.at[pl.ds(r0, RPT)], tB.at[pl.ds(r0, RPT)])
    pltpu.sync_copy(z6_hbm.at[pl.ds(r0, RPT)], tC.at[pl.ds(r0, RPT)])
    plsc.subcore_barrier()

    ebase = wid * EW
    lane = lax.iota(jnp.int32, L)

    def body(i, carry):
        base = ebase + i * CHUNK
        pltpu.sync_copy(rij_hbm.at[pl.ds(base, CHUNK)], rij_v)
        pltpu.sync_copy(src_hbm.at[pl.ds(base, CHUNK)], src_v)
        pltpu.sync_copy(dst_hbm.at[pl.ds(base, CHUNK)], dst_v)
        c0 = jnp.zeros((L,), jnp.int32)
        for j in range(CHUNK // L):
            e = lane + j * L
            x = plsc.load_gather(rij_v, [e, c0])
            y = plsc.load_gather(rij_v, [e, c0 + 1])
            z = plsc.load_gather(rij_v, [e, c0 + 2])
            plsc.store_scatter(vir_v, [e, c0], x * x)
            plsc.store_scatter(vir_v, [e, c0 + 1], y * y)
            plsc.store_scatter(vir_v, [e, c0 + 2], z * z)
            plsc.store_scatter(vir_v, [e, c0 + 3], x * y)
            plsc.store_scatter(vir_v, [e, c0 + 4], y * z)
            plsc.store_scatter(vir_v, [e, c0 + 5], z * x)
        pltpu.sync_copy(rij_v, tA.at[src_v], add=True)
        pltpu.sync_copy(rij_v, tB.at[dst_v], add=True)
        pltpu.sync_copy(vir_v, tC.at[dst_v], add=True)
        return carry

    lax.fori_loop(0, NCHUNK, body, 0)
    plsc.subcore_barrier()

    o0 = cid * NPAD + r0
    pltpu.sync_copy(tA.at[pl.ds(r0, RPT)], outA.at[pl.ds(o0, RPT)])
    pltpu.sync_copy(tB.at[pl.ds(r0, RPT)], outB.at[pl.ds(o0, RPT)])
    pltpu.sync_copy(tC.at[pl.ds(r0, RPT)], outC.at[pl.ds(o0, RPT)])


NBLK = 8
RB = N_NODES // NBLK
FB = 3 * RB


def _combine_body(a_ref, b_ref, posf_ref, ct_ref, batch_ref, cellf_ref,
                  vol_ref, forcef_ref, stress_ref, acc_ref):
    i = pl.program_id(0)

    @pl.when(i == 0)
    def _():
        acc_ref[...] = jnp.zeros_like(acc_ref)

    forcef_ref[...] = (a_ref[0] + a_ref[1] - b_ref[0] - b_ref[1]
                       - 0.2 * posf_ref[...])

    s6 = ct_ref[0] + ct_ref[1]
    bvec = batch_ref[...]
    onehot = (bvec[None, :] == lax.iota(jnp.int32, NBATCH)[:, None])
    contrib = jax.lax.dot_general(
        onehot.astype(jnp.float32), s6,
        dimension_numbers=(((1,), (1,)), ((), ())),
        preferred_element_type=jnp.float32)
    acc_ref[...] = acc_ref[...] + contrib

    @pl.when(i == NBLK - 1)
    def _():
        cf = cellf_ref[...]
        vol = vol_ref[...]
        scale = -0.02 / vol
        cols = []
        for (ii, jj) in ((0, 0), (1, 1), (2, 2), (0, 1), (1, 2), (0, 2)):
            s = (cf[:, ii] * cf[:, jj] + cf[:, 3 + ii] * cf[:, 3 + jj]
                 + cf[:, 6 + ii] * cf[:, 6 + jj])
            cols.append((s * scale)[:, None])
        lr_voigt = jnp.concatenate(cols, axis=-1)
        stress_ref[...] = -acc_ref[...] / vol[:, None] + lr_voigt


def _combine(a2, b2, posf, ct, batch, cellf, vol, interpret=False):
    return pl.pallas_call(
        _combine_body,
        grid=(NBLK,),
        in_specs=[
            pl.BlockSpec((2, FB), lambda i: (0, i)),
            pl.BlockSpec((2, FB), lambda i: (0, i)),
            pl.BlockSpec((FB,), lambda i: (i,)),
            pl.BlockSpec((2, 6, RB), lambda i: (0, 0, i)),
            pl.BlockSpec((RB,), lambda i: (i,)),
            pl.BlockSpec((NBATCH, 9), lambda i: (0, 0)),
            pl.BlockSpec((NBATCH,), lambda i: (0,)),
        ],
        out_specs=[
            pl.BlockSpec((FB,), lambda i: (i,)),
            pl.BlockSpec((NBATCH, 6), lambda i: (0, 0)),
        ],
        out_shape=[
            jax.ShapeDtypeStruct((3 * N_NODES,), jnp.float32),
            jax.ShapeDtypeStruct((NBATCH, 6), jnp.float32),
        ],
        scratch_shapes=[pltpu.VMEM((NBATCH, 6), jnp.float32)],
        compiler_params=pltpu.CompilerParams(
            dimension_semantics=("arbitrary",)),
        interpret=interpret,
    )(a2, b2, posf, ct, batch, cellf, vol)


@jax.jit
def kernel(rij, edge_idx, pos, les_cell, batch, cell_volume, num_atoms):
    del num_atoms
    src = edge_idx[0].astype(jnp.int32)
    dst = edge_idx[1].astype(jnp.int32)
    z3 = jnp.zeros((NPAD, 3), jnp.float32)
    z6 = jnp.zeros((NPAD, 6), jnp.float32)

    outA, outB, outC = _sc_scatter(rij, src, dst, z3, z6)

    a2 = outA.reshape(NC, NPAD, 3)[:, :N_NODES].reshape(NC, 3 * N_NODES)
    b2 = outB.reshape(NC, NPAD, 3)[:, :N_NODES].reshape(NC, 3 * N_NODES)
    ct = outC.reshape(NC, NPAD, 6)[:, :N_NODES].transpose(0, 2, 1)
    posf = pos.reshape(3 * N_NODES)
    cellf = les_cell.reshape(NBATCH, 9)

    forcef, stress = _combine(a2, b2, posf, ct, batch.astype(jnp.int32),
                              cellf, cell_volume)
    return forcef.reshape(N_NODES, 3), stress


# speedup vs baseline: 5.5458x; 5.5458x over previous
import functools

import jax
import jax.numpy as jnp
from jax import lax
from jax.experimental import pallas as pl
from jax.experimental.pallas import tpu as pltpu
from jax.experimental.pallas import tpu_sc as plsc

N_NODES = 100000
N_EDGES = 6400000
NBATCH = 16

NC = 2
NS = 16
L = 16
NW = NC * NS
EW = N_EDGES // NW
CHUNK = 80
NCHUNK = EW // CHUNK

NPAD = 106496
RPT = NPAD // NS
NT = 12


@functools.cache
def _build_sc_scatter():
    return functools.partial(
        pl.kernel,
        out_type=jax.ShapeDtypeStruct((NC * NT * NPAD,), jnp.float32),
        mesh=plsc.VectorSubcoreMesh(core_axis_name="c", subcore_axis_name="s"),
        scratch_types=[
            pltpu.VMEM((CHUNK,), jnp.int32),
            pltpu.VMEM((CHUNK,), jnp.int32),
            pltpu.VMEM((CHUNK,), jnp.float32),
            pltpu.VMEM((CHUNK,), jnp.float32),
            pltpu.VMEM((CHUNK,), jnp.float32),
            [pltpu.VMEM((CHUNK,), jnp.float32) for _ in range(6)],
            pltpu.VMEM((RPT,), jnp.float32),
            [pltpu.VMEM_SHARED((NPAD,), jnp.float32) for _ in range(NT)],
        ],
    )(_sc_scatter_body)


def _sc_scatter_body(x_hbm, y_hbm, z_hbm, src_hbm, dst_hbm, z1_hbm, out_hbm,
                     src_v, dst_v, x_v, y_v, z_v, p_v, stage_v, tbl):
    cid = lax.axis_index("c")
    sid = lax.axis_index("s")
    wid = sid * NC + cid

    r0 = sid * RPT
    pltpu.sync_copy(z1_hbm.at[pl.ds(r0, RPT)], stage_v)
    for t in range(NT):
        pltpu.sync_copy(stage_v, tbl[t].at[pl.ds(r0, RPT)])
    plsc.subcore_barrier()

    ebase = wid * EW

    def body(i, carry):
        base = ebase + i * CHUNK
        pltpu.sync_copy(x_hbm.at[pl.ds(base, CHUNK)], x_v)
        pltpu.sync_copy(y_hbm.at[pl.ds(base, CHUNK)], y_v)
        pltpu.sync_copy(z_hbm.at[pl.ds(base, CHUNK)], z_v)
        pltpu.sync_copy(src_hbm.at[pl.ds(base, CHUNK)], src_v)
        pltpu.sync_copy(dst_hbm.at[pl.ds(base, CHUNK)], dst_v)
        for j in range(CHUNK // L):
            sl = pl.ds(j * L, L)
            x = x_v[sl]
            y = y_v[sl]
            z = z_v[sl]
            p_v[0][sl] = x * x
            p_v[1][sl] = y * y
            p_v[2][sl] = z * z
            p_v[3][sl] = x * y
            p_v[4][sl] = y * z
            p_v[5][sl] = z * x
        pltpu.sync_copy(x_v, tbl[0].at[src_v], add=True)
        pltpu.sync_copy(y_v, tbl[1].at[src_v], add=True)
        pltpu.sync_copy(z_v, tbl[2].at[src_v], add=True)
        pltpu.sync_copy(x_v, tbl[3].at[dst_v], add=True)
        pltpu.sync_copy(y_v, tbl[4].at[dst_v], add=True)
        pltpu.sync_copy(z_v, tbl[5].at[dst_v], add=True)
        for k in range(6):
            pltpu.sync_copy(p_v[k], tbl[6 + k].at[dst_v], add=True)
        return carry

    lax.fori_loop(0, NCHUNK, body, 0)
    plsc.subcore_barrier()

    for t in range(NT):
        pltpu.sync_copy(tbl[t].at[pl.ds(r0, RPT)], stage_v)
        o0 = (cid * NT + t) * NPAD + r0
        pltpu.sync_copy(stage_v, out_hbm.at[pl.ds(o0, RPT)])


NBLK = 8
RB = NPAD // NBLK


def _combine_body(t_ref, post_ref, batch_ref, cellf_ref, vol_ref,
                  forcet_ref, stress_ref, acc_ref):
    i = pl.program_id(0)

    @pl.when(i == 0)
    def _():
        acc_ref[...] = jnp.zeros_like(acc_ref)

    forcet_ref[...] = (t_ref[0, 0:3] + t_ref[1, 0:3]
                       - t_ref[0, 3:6] - t_ref[1, 3:6]
                       - 0.2 * post_ref[...])

    s6 = t_ref[0, 6:12] + t_ref[1, 6:12]
    bvec = batch_ref[...]
    onehot = (bvec[None, :] == lax.iota(jnp.int32, NBATCH)[:, None])
    contrib = jax.lax.dot_general(
        onehot.astype(jnp.float32), s6,
        dimension_numbers=(((1,), (1,)), ((), ())),
        preferred_element_type=jnp.float32)
    acc_ref[...] = acc_ref[...] + contrib

    @pl.when(i == NBLK - 1)
    def _():
        cf = cellf_ref[...]
        vol = vol_ref[...]
        scale = -0.02 / vol
        cols = []
        for (ii, jj) in ((0, 0), (1, 1), (2, 2), (0, 1), (1, 2), (0, 2)):
            s = (cf[:, ii] * cf[:, jj] + cf[:, 3 + ii] * cf[:, 3 + jj]
                 + cf[:, 6 + ii] * cf[:, 6 + jj])
            cols.append((s * scale)[:, None])
        lr_voigt = jnp.concatenate(cols, axis=-1)
        stress_ref[...] = -acc_ref[...] / vol[:, None] + lr_voigt


def _combine(t2, post, batch, cellf, vol, interpret=False):
    return pl.pallas_call(
        _combine_body,
        grid=(NBLK,),
        in_specs=[
            pl.BlockSpec((2, NT, RB), lambda i: (0, 0, i)),
            pl.BlockSpec((3, RB), lambda i: (0, i)),
            pl.BlockSpec((RB,), lambda i: (i,)),
            pl.BlockSpec((NBATCH, 9), lambda i: (0, 0)),
            pl.BlockSpec((NBATCH,), lambda i: (0,)),
        ],
        out_specs=[
            pl.BlockSpec((3, RB), lambda i: (0, i)),
            pl.BlockSpec((NBATCH, 6), lambda i: (0, 0)),
        ],
        out_shape=[
            jax.ShapeDtypeStruct((3, NPAD), jnp.float32),
            jax.ShapeDtypeStruct((NBATCH, 6), jnp.float32),
        ],
        scratch_shapes=[pltpu.VMEM((NBATCH, 6), jnp.float32)],
        compiler_params=pltpu.CompilerParams(
            dimension_semantics=("arbitrary",)),
        interpret=interpret,
    )(t2, post, batch, cellf, vol)


@jax.jit
def kernel(rij, edge_idx, pos, les_cell, batch, cell_volume, num_atoms):
    del num_atoms
    src = edge_idx[0].astype(jnp.int32)
    dst = edge_idx[1].astype(jnp.int32)
    xp = rij[:, 0]
    yp = rij[:, 1]
    zp = rij[:, 2]
    z1 = jnp.zeros((NPAD,), jnp.float32)

    out = _build_sc_scatter()(xp, yp, zp, src, dst, z1)

    t2 = out.reshape(NC, NT, NPAD)
    post = jnp.pad(pos.T, ((0, 0), (0, NPAD - N_NODES)))
    batch_p = jnp.pad(batch.astype(jnp.int32), (0, NPAD - N_NODES))
    cellf = les_cell.reshape(NBATCH, 9)

    forcet, stress = _combine(t2, post, batch_p, cellf, cell_volume)
    return forcet[:, :N_NODES].T, stress


# speedup vs baseline: 41.7476x; 7.5278x over previous
import functools

import jax
import jax.numpy as jnp
from jax import lax
from jax.experimental import pallas as pl
from jax.experimental.pallas import tpu as pltpu
from jax.experimental.pallas import tpu_sc as plsc

N_NODES = 100000
N_EDGES = 6400000
NBATCH = 16

NC = 2
NS = 16
L = 16
NW = NC * NS
EW = N_EDGES // NW
CHUNK = 128
NCHUNK = 1564
EWP = CHUNK * NCHUNK
EPAD = EWP * NW
NBUF = 4

NPAD = 106496
RPT = NPAD // NS
NT = 12


@functools.cache
def _build_sc_scatter():
    return functools.partial(
        pl.kernel,
        out_type=jax.ShapeDtypeStruct((NC * NT * NPAD,), jnp.float32),
        mesh=plsc.VectorSubcoreMesh(core_axis_name="c", subcore_axis_name="s"),
        scratch_types=[
            [pltpu.VMEM((CHUNK,), jnp.int32) for _ in range(NBUF)],
            [pltpu.VMEM((CHUNK,), jnp.int32) for _ in range(NBUF)],
            [pltpu.VMEM((CHUNK,), jnp.float32) for _ in range(NBUF)],
            [pltpu.VMEM((CHUNK,), jnp.float32) for _ in range(NBUF)],
            [pltpu.VMEM((CHUNK,), jnp.float32) for _ in range(NBUF)],
            [[pltpu.VMEM((CHUNK,), jnp.float32) for _ in range(6)]
             for _ in range(NBUF)],
            pltpu.VMEM((RPT,), jnp.float32),
            [pltpu.VMEM_SHARED((NPAD,), jnp.float32) for _ in range(NT)],
            [pltpu.SemaphoreType.DMA for _ in range(NBUF)],
            [pltpu.SemaphoreType.DMA for _ in range(NBUF)],
        ],
    )(_sc_scatter_body)


def _sc_scatter_body(x_hbm, y_hbm, z_hbm, src_hbm, dst_hbm, z1_hbm, out_hbm,
                     src_v, dst_v, x_v, y_v, z_v, p_v, stage_v, tbl,
                     lsem, ssem):
    cid = lax.axis_index("c")
    sid = lax.axis_index("s")
    wid = sid * NC + cid

    r0 = sid * RPT
    pltpu.sync_copy(z1_hbm.at[pl.ds(r0, RPT)], stage_v)
    for t in range(NT):
        pltpu.sync_copy(stage_v, tbl[t].at[pl.ds(r0, RPT)])
    plsc.subcore_barrier()

    ebase = wid * EWP

    def fire_loads(g, b):
        base = ebase + g * CHUNK
        pltpu.async_copy(x_hbm.at[pl.ds(base, CHUNK)], x_v[b], lsem[b])
        pltpu.async_copy(y_hbm.at[pl.ds(base, CHUNK)], y_v[b], lsem[b])
        pltpu.async_copy(z_hbm.at[pl.ds(base, CHUNK)], z_v[b], lsem[b])
        pltpu.async_copy(src_hbm.at[pl.ds(base, CHUNK)], src_v[b], lsem[b])
        pltpu.async_copy(dst_hbm.at[pl.ds(base, CHUNK)], dst_v[b], lsem[b])

    def wait_loads(b):
        for r in (x_v[b], y_v[b], z_v[b]):
            pltpu.make_async_copy(x_hbm.at[pl.ds(0, CHUNK)], r, lsem[b]).wait()
        for r in (src_v[b], dst_v[b]):
            pltpu.make_async_copy(src_hbm.at[pl.ds(0, CHUNK)], r, lsem[b]).wait()

    def fire_scatters(b):
        pltpu.async_copy(x_v[b], tbl[0].at[src_v[b]], ssem[b], add=True)
        pltpu.async_copy(y_v[b], tbl[1].at[src_v[b]], ssem[b], add=True)
        pltpu.async_copy(z_v[b], tbl[2].at[src_v[b]], ssem[b], add=True)
        pltpu.async_copy(x_v[b], tbl[3].at[dst_v[b]], ssem[b], add=True)
        pltpu.async_copy(y_v[b], tbl[4].at[dst_v[b]], ssem[b], add=True)
        pltpu.async_copy(z_v[b], tbl[5].at[dst_v[b]], ssem[b], add=True)
        for k in range(6):
            pltpu.async_copy(p_v[b][k], tbl[6 + k].at[dst_v[b]], ssem[b],
                             add=True)

    def wait_scatters(b):
        for _ in range(NT):
            pltpu.make_async_copy(x_v[b], tbl[0].at[src_v[b]], ssem[b]).wait()

    def compute(b):
        for j in range(CHUNK // L):
            sl = pl.ds(j * L, L)
            x = x_v[b][sl]
            y = y_v[b][sl]
            z = z_v[b][sl]
            p_v[b][0][sl] = x * x
            p_v[b][1][sl] = y * y
            p_v[b][2][sl] = z * z
            p_v[b][3][sl] = x * y
            p_v[b][4][sl] = y * z
            p_v[b][5][sl] = z * x

    fire_loads(0, 0)
    fire_loads(1, 1)

    def body(i, carry):
        for b in range(NBUF):
            g = i * NBUF + b
            bn = (b + 2) % NBUF
            if b < 2:
                @pl.when(i > 0)
                def _():
                    wait_scatters(bn)

                fire_loads(g + 2, bn)
            else:
                wait_scatters(bn)

                @pl.when(g + 2 < NCHUNK)
                def _():
                    fire_loads(g + 2, bn)
            wait_loads(b)
            compute(b)
            fire_scatters(b)
        return carry

    lax.fori_loop(0, NCHUNK // NBUF, body, 0)
    wait_scatters((NCHUNK - 2) % NBUF)
    wait_scatters((NCHUNK - 1) % NBUF)
    plsc.subcore_barrier()

    for t in range(NT):
        pltpu.sync_copy(tbl[t].at[pl.ds(r0, RPT)], stage_v)
        o0 = (cid * NT + t) * NPAD + r0
        pltpu.sync_copy(stage_v, out_hbm.at[pl.ds(o0, RPT)])


NBLK = 8
RB = NPAD // NBLK


def _combine_body(t_ref, post_ref, batch_ref, cellf_ref, vol_ref,
                  forcet_ref, stress_ref, acc_ref):
    i = pl.program_id(0)

    @pl.when(i == 0)
    def _():
        acc_ref[...] = jnp.zeros_like(acc_ref)

    forcet_ref[...] = (t_ref[0, 0:3] + t_ref[1, 0:3]
                       - t_ref[0, 3:6] - t_ref[1, 3:6]
                       - 0.2 * post_ref[...])

    s6 = t_ref[0, 6:12] + t_ref[1, 6:12]
    bvec = batch_ref[...]
    onehot = (bvec[None, :] == lax.iota(jnp.int32, NBATCH)[:, None])
    contrib = jax.lax.dot_general(
        onehot.astype(jnp.float32), s6,
        dimension_numbers=(((1,), (1,)), ((), ())),
        preferred_element_type=jnp.float32)
    acc_ref[...] = acc_ref[...] + contrib

    @pl.when(i == NBLK - 1)
    def _():
        cf = cellf_ref[...]
        vol = vol_ref[...]
        scale = -0.02 / vol
        cols = []
        for (ii, jj) in ((0, 0), (1, 1), (2, 2), (0, 1), (1, 2), (0, 2)):
            s = (cf[:, ii] * cf[:, jj] + cf[:, 3 + ii] * cf[:, 3 + jj]
                 + cf[:, 6 + ii] * cf[:, 6 + jj])
            cols.append((s * scale)[:, None])
        lr_voigt = jnp.concatenate(cols, axis=-1)
        stress_ref[...] = -acc_ref[...] / vol[:, None] + lr_voigt


def _combine(t2, post, batch, cellf, vol, interpret=False):
    return pl.pallas_call(
        _combine_body,
        grid=(NBLK,),
        in_specs=[
            pl.BlockSpec((2, NT, RB), lambda i: (0, 0, i)),
            pl.BlockSpec((3, RB), lambda i: (0, i)),
            pl.BlockSpec((RB,), lambda i: (i,)),
            pl.BlockSpec((NBATCH, 9), lambda i: (0, 0)),
            pl.BlockSpec((NBATCH,), lambda i: (0,)),
        ],
        out_specs=[
            pl.BlockSpec((3, RB), lambda i: (0, i)),
            pl.BlockSpec((NBATCH, 6), lambda i: (0, 0)),
        ],
        out_shape=[
            jax.ShapeDtypeStruct((3, NPAD), jnp.float32),
            jax.ShapeDtypeStruct((NBATCH, 6), jnp.float32),
        ],
        scratch_shapes=[pltpu.VMEM((NBATCH, 6), jnp.float32)],
        compiler_params=pltpu.CompilerParams(
            dimension_semantics=("arbitrary",)),
        interpret=interpret,
    )(t2, post, batch, cellf, vol)


@jax.jit
def kernel(rij, edge_idx, pos, les_cell, batch, cell_volume, num_atoms):
    del num_atoms
    pe = EPAD - N_EDGES
    src = jnp.pad(edge_idx[0].astype(jnp.int32), (0, pe))
    dst = jnp.pad(edge_idx[1].astype(jnp.int32), (0, pe))
    xp = jnp.pad(rij[:, 0], (0, pe))
    yp = jnp.pad(rij[:, 1], (0, pe))
    zp = jnp.pad(rij[:, 2], (0, pe))
    z1 = jnp.zeros((NPAD,), jnp.float32)

    out = _build_sc_scatter()(xp, yp, zp, src, dst, z1)

    t2 = out.reshape(NC, NT, NPAD)
    post = jnp.pad(pos.T, ((0, 0), (0, NPAD - N_NODES)))
    batch_p = jnp.pad(batch.astype(jnp.int32), (0, NPAD - N_NODES))
    cellf = les_cell.reshape(NBATCH, 9)

    forcet, stress = _combine(t2, post, batch_p, cellf, cell_volume)
    return forcet[:, :N_NODES].T, stress
